# baseline (speedup 1.0000x reference)
import functools

import jax
import jax.numpy as jnp
from jax import lax
from jax.experimental import pallas as pl
from jax.experimental.pallas import tpu as pltpu
from jax.experimental.pallas import tpu_sc as plsc

NO_TOKENS = 1000000
EMBED_DIM = 64

_INFO = plsc.get_sparse_core_info()
_NC, _NS = _INFO.num_cores, _INFO.num_subcores
_NW = _NC * _NS

_CHUNK = 128


def _gather_kernel(b_per_w, steps, idx_hbm, table_hbm, out_hbm,
                   idx_v, rows_v, gsem):
  wid = lax.axis_index("s") * _NC + lax.axis_index("c")
  base = wid * b_per_w
  pltpu.sync_copy(idx_hbm.at[pl.ds(base, b_per_w)], idx_v)

  def body(g, carry):
    off = pl.multiple_of(g * _CHUNK, _CHUNK)
    idx_slice = idx_v.at[pl.ds(off, _CHUNK)]
    pltpu.async_copy(table_hbm.at[idx_slice], rows_v, gsem).wait()
    pltpu.sync_copy(rows_v, out_hbm.at[pl.ds(base + off, _CHUNK)])
    return carry

  lax.fori_loop(0, steps, body, 0)


@jax.jit
def kernel(input, table):
  orig_shape = input.shape
  flat_idx = input.reshape(-1).astype(jnp.int32)
  n = flat_idx.shape[0]
  assert n % (_NW * _CHUNK) == 0
  b_per_w = n // _NW
  steps = b_per_w // _CHUNK

  mesh = plsc.VectorSubcoreMesh(core_axis_name="c", subcore_axis_name="s")
  out = pl.kernel(
      functools.partial(_gather_kernel, b_per_w, steps),
      out_type=jax.ShapeDtypeStruct((n, EMBED_DIM), jnp.float32),
      mesh=mesh,
      scratch_types=[
          pltpu.VMEM((b_per_w,), jnp.int32),
          pltpu.VMEM((_CHUNK, EMBED_DIM), jnp.float32),
          pltpu.SemaphoreType.DMA,
      ],
      compiler_params=pltpu.CompilerParams(use_tc_tiling_on_sc=False),
  )(flat_idx, table)
  return out.reshape(*orig_shape, EMBED_DIM)


# speedup vs baseline: 1.1170x; 1.1170x over previous
import functools

import jax
import jax.numpy as jnp
from jax import lax
from jax.experimental import pallas as pl
from jax.experimental.pallas import tpu as pltpu
from jax.experimental.pallas import tpu_sc as plsc

EMBED_DIM = 64

_INFO = plsc.get_sparse_core_info()
_NC, _NS = _INFO.num_cores, _INFO.num_subcores
_NW = _NC * _NS

_NT = 128
_CH = 256
_NBUF = 4


def _gather_kernel(b_per_w, T, idx_hbm, table_hbm, out_hbm, idx_v,
                   b0, b1, b2, b3, g0, g1, g2, g3, w0, w1, w2, w3):
  bufs = (b0, b1, b2, b3)
  gsems = (g0, g1, g2, g3)
  wsems = (w0, w1, w2, w3)
  wid = lax.axis_index("s") * _NC + lax.axis_index("c")
  base = wid * b_per_w
  pltpu.sync_copy(idx_hbm.at[pl.ds(base, b_per_w)], idx_v)

  def fire(t, b):
    for j in range(_CH // _NT):
      off = pl.multiple_of(t * _CH + j * _NT, _NT)
      pltpu.async_copy(table_hbm.at[idx_v.at[pl.ds(off, _NT)]],
                       bufs[b].at[pl.ds(j * _NT, _NT)], gsems[b])

  def wait_g(b):
    pltpu.make_async_copy(table_hbm.at[pl.ds(0, _CH)], bufs[b],
                          gsems[b]).wait()

  def awrite(t, b):
    off = pl.multiple_of(base + t * _CH, _NT)
    pltpu.async_copy(bufs[b], out_hbm.at[pl.ds(off, _CH)], wsems[b])

  def wait_w(b):
    pltpu.make_async_copy(bufs[b], out_hbm.at[pl.ds(0, _CH)],
                          wsems[b]).wait()

  fire(0, 0)
  fire(1, 1)
  wait_g(0); awrite(0, 0); fire(2, 2)
  wait_g(1); awrite(1, 1); fire(3, 3)

  @pl.loop(2, T - 2, step=_NBUF)
  def _main(t0):
    for i in range(_NBUF):
      t = t0 + i
      b = (2 + i) % _NBUF
      wait_g(b)
      awrite(t, b)
      wait_w((b + 2) % _NBUF)
      fire(t + 2, (b + 2) % _NBUF)

  wait_g(2); awrite(T - 2, 2); wait_w(0)
  wait_g(3); awrite(T - 1, 3); wait_w(1)
  wait_w(2); wait_w(3)


@jax.jit
def kernel(input, table):
  orig_shape = input.shape
  flat_idx = input.reshape(-1).astype(jnp.int32)
  n = flat_idx.shape[0]
  assert n % (_NW * _CH) == 0
  b_per_w = n // _NW
  T = b_per_w // _CH
  assert (T - 4) % _NBUF == 0

  mesh = plsc.VectorSubcoreMesh(core_axis_name="c", subcore_axis_name="s")
  out = pl.kernel(
      functools.partial(_gather_kernel, b_per_w, T),
      out_type=jax.ShapeDtypeStruct((n, EMBED_DIM), jnp.float32),
      mesh=mesh,
      scratch_types=[pltpu.VMEM((b_per_w,), jnp.int32)]
      + [pltpu.VMEM((_CH, EMBED_DIM), jnp.float32) for _ in range(_NBUF)]
      + [pltpu.SemaphoreType.DMA for _ in range(2 * _NBUF)],
      compiler_params=pltpu.CompilerParams(use_tc_tiling_on_sc=False),
  )(flat_idx, table)
  return out.reshape(*orig_shape, EMBED_DIM)
